# baseline (speedup 1.0000x reference)
import functools

import jax
import jax.numpy as jnp
from jax import lax
from jax.experimental import pallas as pl
from jax.experimental.pallas import tpu as pltpu
from jax.experimental.pallas import tpu_sc as plsc

N = 10000
NP = 10112
D = 128
E = 320000
NEG = 0.2
BN = 400
NB = N // BN
KCH = 128
HC = 64
NTILES = 16
ROW_T = 640
ROW_LAST = N - 15 * ROW_T
EPT8 = 156 * KCH
EPT1 = 78 * KCH
EPT_W8 = 624 * KCH


def _feat_el_er(hp, W, al3, ar3, H_in, H_out):
    def body(hp_ref, w_ref, al_ref, ar_ref, feat_ref, el_ref, er_ref):
        el_cols, er_cols = [], []
        for ho in range(H_out):
            f_h = hp_ref[0] @ w_ref[0:128, ho * 128:(ho + 1) * 128]
            for hi in range(1, H_in):
                f_h = f_h + hp_ref[hi] @ w_ref[hi * 128:(hi + 1) * 128,
                                               ho * 128:(ho + 1) * 128]
            feat_ref[ho] = f_h
            el_cols.append(f_h @ al_ref[ho])
            er_cols.append(f_h @ ar_ref[ho])
        el_ref[...] = (jnp.concatenate(el_cols, axis=1)
                       if H_out > 1 else el_cols[0])
        er_ref[...] = (jnp.concatenate(er_cols, axis=1)
                       if H_out > 1 else er_cols[0])

    return pl.pallas_call(
        body,
        grid=(NB,),
        in_specs=[
            pl.BlockSpec((H_in, BN, D), lambda i: (0, i, 0)),
            pl.BlockSpec((H_in * D, H_out * D), lambda i: (0, 0)),
            pl.BlockSpec((H_out, D, 1), lambda i: (0, 0, 0)),
            pl.BlockSpec((H_out, D, 1), lambda i: (0, 0, 0)),
        ],
        out_specs=[
            pl.BlockSpec((H_out, BN, D), lambda i: (0, i, 0)),
            pl.BlockSpec((BN, H_out), lambda i: (i, 0)),
            pl.BlockSpec((BN, H_out), lambda i: (i, 0)),
        ],
        out_shape=[
            jax.ShapeDtypeStruct((H_out, N, D), jnp.float32),
            jax.ShapeDtypeStruct((N, H_out), jnp.float32),
            jax.ShapeDtypeStruct((N, H_out), jnp.float32),
        ],
    )(hp, W, al3, ar3)


def _norm_act(out_acc, denT, H):
    def body(o_ref, d_ref, y_ref):
        for h in range(H):
            dn = d_ref[:, h:h + 1]
            safe = jnp.where(dn == 0.0, 1.0, dn)
            x = o_ref[h] / safe
            y_ref[h] = jnp.where(x > 0.0, x, jnp.exp(x) - 1.0)

    return pl.pallas_call(
        body,
        grid=(NB,),
        in_specs=[
            pl.BlockSpec((H, BN, D), lambda i: (0, i, 0)),
            pl.BlockSpec((BN, H), lambda i: (i, 0)),
        ],
        out_specs=pl.BlockSpec((H, BN, D), lambda i: (0, i, 0)),
        out_shape=jax.ShapeDtypeStruct((H, N, D), jnp.float32),
    )(out_acc, denT)


def _final(out2, den2T, Wc, bc2):
    NC = Wc.shape[1]

    def body(o_ref, d_ref, wc_ref, bc_ref, log_ref, h3_ref):
        s = o_ref[0] + o_ref[1]
        dn = d_ref[:, 0:1] + d_ref[:, 1:2]
        safe = jnp.where(dn == 0.0, 1.0, dn)
        h3 = s / safe
        h3_ref[...] = h3
        log_ref[...] = h3 @ wc_ref[...] + bc_ref[...]

    return pl.pallas_call(
        body,
        grid=(NB,),
        in_specs=[
            pl.BlockSpec((2, BN, D), lambda i: (0, i, 0)),
            pl.BlockSpec((BN, 2), lambda i: (i, 0)),
            pl.BlockSpec((D, NC), lambda i: (0, 0)),
            pl.BlockSpec((1, NC), lambda i: (0, 0)),
        ],
        out_specs=[
            pl.BlockSpec((BN, NC), lambda i: (i, 0)),
            pl.BlockSpec((BN, D), lambda i: (i, 0)),
        ],
        out_shape=[
            jax.ShapeDtypeStruct((N, NC), jnp.float32),
            jax.ShapeDtypeStruct((N, D), jnp.float32),
        ],
    )(out2, den2T, Wc, bc2)


def _sc_w(Htot):
    if Htot > 1:
        heads_per_sc = Htot // 2
        n_out = Htot
    else:
        heads_per_sc = 1
        n_out = 2
    mesh = plsc.VectorSubcoreMesh(core_axis_name="c", subcore_axis_name="s")

    @functools.partial(
        pl.kernel,
        out_type=(
            jax.ShapeDtypeStruct((Htot, E), jnp.float32),
            jax.ShapeDtypeStruct((n_out, NP), jnp.float32),
        ),
        mesh=mesh,
        compiler_params=pltpu.CompilerParams(needs_layout_passes=False),
        scratch_types=[
            pltpu.VMEM((NP,), jnp.float32),
            pltpu.VMEM((NP,), jnp.float32),
            pltpu.VMEM((KCH,), jnp.int32),
            pltpu.VMEM((KCH,), jnp.int32),
            pltpu.VMEM((KCH,), jnp.int32),
            pltpu.VMEM((KCH,), jnp.int32),
            pltpu.VMEM((KCH,), jnp.float32),
            pltpu.VMEM((KCH,), jnp.float32),
            pltpu.VMEM((KCH,), jnp.int32),
            pltpu.VMEM((KCH,), jnp.int32),
            pltpu.VMEM((128,), jnp.float32),
            pltpu.VMEM_SHARED((heads_per_sc * NP,), jnp.float32),
            pltpu.SemaphoreType.DMA,
            pltpu.SemaphoreType.DMA,
            pltpu.SemaphoreType.DMA,
        ],
    )
    def k(src_h, dst_h, elT_h, erT_h, w_hbm, den_h,
          el_v, er_v, srcb0, srcb1, dstb0, dstb1, wb0, wb1, dn0, dn1, zd,
          den_sp, sem_is, sem_id, sem_w):
        c = lax.axis_index("c")
        s = lax.axis_index("s")
        wid = c * NTILES + s
        zero16 = jnp.zeros((16,), jnp.float32)
        ibufs = ((srcb0, dstb0, wb0, dn0), (srcb1, dstb1, wb1, dn1))
        if Htot > 1:
            lh = s // 4
            h_ix = c * heads_per_sc + lh
            ebase = (s % 4) * EPT_W8
            nch = jnp.where(s % 4 == 3, 628, 624)
            nzc, nzc_last = 20, 16
        else:
            lh = 0
            h_ix = 0
            ebase = wid * EPT1
            nch = jnp.where(wid == 31, 82, 78)
            nzc, nzc_last = 5, 4

        for j8 in range(8):
            zd[pl.ds(j8 * 16, 16)] = zero16

        @pl.when(s < 15)
        def _():
            for j in range(nzc):
                pltpu.sync_copy(
                    zd, den_sp.at[pl.ds((s * nzc + j) * 128, 128)])

        @pl.when(s == 15)
        def _():
            for j in range(nzc_last):
                pltpu.sync_copy(
                    zd, den_sp.at[pl.ds((15 * nzc + j) * 128, 128)])

        pltpu.sync_copy(elT_h.at[h_ix], el_v)
        pltpu.sync_copy(erT_h.at[h_ix], er_v)
        plsc.subcore_barrier()
        pltpu.sync_copy(src_h.at[pl.ds(ebase, KCH)], srcb0)
        pltpu.sync_copy(dst_h.at[pl.ds(ebase, KCH)], dstb0)
        dnoff = lh * NP

        def pair(ip, carry):
            for p in range(2):
                srcb, dstb, wbuf, dnb = ibufs[p]
                srcb_o, dstb_o, _, _ = ibufs[1 - p]
                base = ebase + (ip * 2 + p) * KCH

                @pl.when(ip > 0)
                def _():
                    pltpu.make_async_copy(
                        wbuf, w_hbm.at[h_ix].at[pl.ds(base, KCH)],
                        sem_w).wait()

                def wait_idx():
                    pltpu.make_async_copy(
                        src_h.at[pl.ds(base, KCH)], srcb, sem_is).wait()
                    pltpu.make_async_copy(
                        dst_h.at[pl.ds(base, KCH)], dstb, sem_id).wait()

                if p == 0:
                    @pl.when(ip > 0)
                    def _():
                        wait_idx()
                else:
                    wait_idx()
                nxt = jnp.minimum(base + KCH, E - KCH)
                pltpu.async_copy(src_h.at[pl.ds(nxt, KCH)], srcb_o, sem_is)
                pltpu.async_copy(dst_h.at[pl.ds(nxt, KCH)], dstb_o, sem_id)
                for j in range(KCH // 16):
                    sl = pl.ds(j * 16, 16)
                    s16 = srcb[sl]
                    d16 = dstb[sl]
                    e16 = (plsc.load_gather(el_v, [s16])
                           + plsc.load_gather(er_v, [d16]))
                    e16 = jnp.where(e16 >= 0.0, e16, e16 * NEG)
                    wbuf[sl] = jnp.exp(e16)
                    dnb[sl] = d16 + dnoff
                pltpu.sync_copy(wbuf, den_sp.at[dnb], add=True)
                pltpu.async_copy(wbuf, w_hbm.at[h_ix].at[pl.ds(base, KCH)],
                                 sem_w)
            return carry

        lax.fori_loop(0, nch // 2, pair, 0)
        pltpu.make_async_copy(wb0, w_hbm.at[h_ix].at[pl.ds(ebase, KCH)],
                              sem_w).wait()
        pltpu.make_async_copy(wb1, w_hbm.at[h_ix].at[pl.ds(ebase, KCH)],
                              sem_w).wait()
        pltpu.make_async_copy(src_h.at[pl.ds(ebase, KCH)], srcb0,
                              sem_is).wait()
        pltpu.make_async_copy(dst_h.at[pl.ds(ebase, KCH)], dstb0,
                              sem_id).wait()
        plsc.subcore_barrier()

        @pl.when(s < heads_per_sc)
        def _():
            pltpu.sync_copy(den_sp.at[pl.ds(s * NP, NP)],
                            den_h.at[c * heads_per_sc + s])

    return k


def _sc_edge2(Htot):
    if Htot > 1:
        heads_per_sc = Htot // 2
        n_out = Htot
    else:
        heads_per_sc = 1
        n_out = 2
    mesh = plsc.VectorSubcoreMesh(core_axis_name="c", subcore_axis_name="s")

    @functools.partial(
        pl.kernel,
        out_type=jax.ShapeDtypeStruct((n_out, N, D), jnp.float32),
        mesh=mesh,
        compiler_params=pltpu.CompilerParams(needs_layout_passes=False),
        scratch_types=[
            pltpu.VMEM((KCH,), jnp.int32),
            pltpu.VMEM((KCH,), jnp.int32),
            pltpu.VMEM((KCH,), jnp.int32),
            pltpu.VMEM((KCH,), jnp.int32),
            pltpu.VMEM((KCH,), jnp.float32),
            pltpu.VMEM((KCH,), jnp.float32),
            pltpu.VMEM((HC,), jnp.int32),
            pltpu.VMEM((HC,), jnp.int32),
            pltpu.VMEM((HC,), jnp.int32),
            pltpu.VMEM((HC,), jnp.int32),
            pltpu.VMEM((HC,), jnp.float32),
            pltpu.VMEM((HC,), jnp.float32),
            pltpu.VMEM((HC,), jnp.int32),
            pltpu.VMEM((HC,), jnp.int32),
            pltpu.VMEM((HC,), jnp.int32),
            pltpu.VMEM((HC,), jnp.int32),
            pltpu.VMEM((HC,), jnp.float32),
            pltpu.VMEM((HC,), jnp.float32),
            pltpu.VMEM((HC, D), jnp.float32),
            pltpu.VMEM((HC, D), jnp.float32),
            pltpu.VMEM((HC, D), jnp.float32),
            pltpu.VMEM((HC, D), jnp.float32),
            pltpu.VMEM_SHARED((N, D), jnp.float32),
            pltpu.SemaphoreType.DMA,
            pltpu.SemaphoreType.DMA,
            pltpu.SemaphoreType.DMA,
            pltpu.SemaphoreType.DMA,
            pltpu.SemaphoreType.DMA,
            pltpu.SemaphoreType.DMA,
            pltpu.SemaphoreType.DMA,
            pltpu.SemaphoreType.DMA,
            pltpu.SemaphoreType.DMA,
            pltpu.SemaphoreType.DMA,
            pltpu.SemaphoreType.DMA,
        ],
    )
    def k(src_h, dst_h, w_hbm, featf_h, out_h,
          srcb0, srcb1, dstb0, dstb1, wch0, wch1,
          dba0, dbb0, s2a0, s2b0, wa0, wb0,
          dba1, dbb1, s2a1, s2b1, wa1, wb1,
          ra0, rb0, ra1, rb1, out_sp,
          sga0, sgb0, sga1, sgb1, ssa0, ssb0, ssa1, ssb1,
          sem_is, sem_id, sem_w):
        c = lax.axis_index("c")
        s = lax.axis_index("s")
        zero16 = jnp.zeros((16,), jnp.float32)
        ibufs = ((srcb0, dstb0, wch0), (srcb1, dstb1, wch1))
        bufs = ((dba0, dbb0, s2a0, s2b0, wa0, wb0),
                (dba1, dbb1, s2a1, s2b1, wa1, wb1))
        rows = ((ra0, rb0), (ra1, rb1))
        gsems = ((sga0, sgb0), (sga1, sgb1))
        ssems = ((ssa0, ssb0), (ssa1, ssb1))

        for hh in range(heads_per_sc):
            if Htot > 1:
                h_ix = c * heads_per_sc + hh
                out_ix = h_ix
                ebase = s * EPT8
                nch = jnp.where(s == 15, 160, 156)
            else:
                h_ix = 0
                out_ix = c
                ebase = (c * NTILES + s) * EPT1
                nch = jnp.where(c * NTILES + s == 31, 82, 78)
            hoff = h_ix * N
            wrow = w_hbm.at[h_ix]

            def zb_body(i, carry):
                for k8 in range(8):
                    ra0[i, pl.ds(k8 * 16, 16)] = zero16
                return carry

            lax.fori_loop(0, HC, zb_body, 0)

            @pl.when(s < 15)
            def _():
                for j in range(10):
                    pltpu.sync_copy(
                        ra0, out_sp.at[pl.ds(s * ROW_T + j * HC, HC)])

            @pl.when(s == 15)
            def _():
                for j in range(6):
                    pltpu.sync_copy(
                        ra0, out_sp.at[pl.ds(15 * ROW_T + j * HC, HC)])
                pltpu.sync_copy(ra0.at[pl.ds(0, 16)],
                                out_sp.at[pl.ds(15 * ROW_T + 384, 16)])

            def build(p):
                srcb, dstb, wch = ibufs[p]
                dba, dbb, s2a, s2b, wa, wb = bufs[p]
                for j in range(KCH // 16):
                    sl = pl.ds(j * 16, 16)
                    sl4 = pl.ds((j % 4) * 16, 16)
                    s16 = srcb[sl]
                    d16 = dstb[sl]
                    w16 = wch[sl]
                    if j < 4:
                        s2a[sl4] = s16 + hoff
                        dba[sl4] = d16
                        wa[sl4] = w16
                    else:
                        s2b[sl4] = s16 + hoff
                        dbb[sl4] = d16
                        wb[sl4] = w16

            def issue_gathers(p):
                dba, dbb, s2a, s2b, wa, wb = bufs[p]
                pltpu.async_copy(featf_h.at[s2a], rows[p][0], gsems[p][0])
                pltpu.async_copy(featf_h.at[s2b], rows[p][1], gsems[p][1])

            def prefetch(base_nxt, p):
                srcb, dstb, wch = ibufs[p]
                nxt = jnp.minimum(base_nxt, E - KCH)
                pltpu.async_copy(src_h.at[pl.ds(nxt, KCH)], srcb, sem_is)
                pltpu.async_copy(dst_h.at[pl.ds(nxt, KCH)], dstb, sem_id)
                pltpu.async_copy(wrow.at[pl.ds(nxt, KCH)], wch, sem_w)

            def scale(rbuf, wbuf):
                def sbody(g2, carry2):
                    w16 = wbuf[pl.ds(g2 * 16, 16)]
                    for e in range(16):
                        r = g2 * 16 + e
                        ws = w16[e]
                        for k8 in range(8):
                            csl = pl.ds(k8 * 16, 16)
                            rbuf[r, csl] = rbuf[r, csl] * ws
                    return carry2

                lax.fori_loop(0, HC // 16, sbody, 0)

            pltpu.sync_copy(src_h.at[pl.ds(ebase, KCH)], srcb0)
            pltpu.sync_copy(dst_h.at[pl.ds(ebase, KCH)], dstb0)
            pltpu.sync_copy(wrow.at[pl.ds(ebase, KCH)], wch0)
            build(0)
            issue_gathers(0)
            prefetch(ebase + KCH, 1)
            plsc.subcore_barrier()

            def pair(ip, carry):
                for p in range(2):
                    q = 1 - p
                    base = ebase + (ip * 2 + p) * KCH
                    dba, dbb, s2a, s2b, wa, wb = bufs[p]
                    dbao, dbbo, _, _, _, _ = bufs[q]
                    srcb_q, dstb_q, wch_q = ibufs[q]

                    nb = base + KCH
                    nbc = jnp.minimum(nb, E - KCH)
                    pltpu.make_async_copy(
                        src_h.at[pl.ds(nbc, KCH)], srcb_q, sem_is).wait()
                    pltpu.make_async_copy(
                        dst_h.at[pl.ds(nbc, KCH)], dstb_q, sem_id).wait()
                    pltpu.make_async_copy(
                        wrow.at[pl.ds(nbc, KCH)], wch_q, sem_w).wait()

                    def wait_sc_prev():
                        pltpu.make_async_copy(
                            rows[q][0], out_sp.at[dbao], ssems[q][0]).wait()
                        pltpu.make_async_copy(
                            rows[q][1], out_sp.at[dbbo], ssems[q][1]).wait()

                    if p == 0:
                        @pl.when(ip > 0)
                        def _():
                            wait_sc_prev()
                    else:
                        wait_sc_prev()
                    build(q)
                    issue_gathers(q)
                    prefetch(nb + KCH, p)
                    ga, gb = gsems[p]
                    pltpu.make_async_copy(featf_h.at[s2a], rows[p][0],
                                          ga).wait()
                    scale(rows[p][0], wa)
                    pltpu.async_copy(rows[p][0], out_sp.at[dba], ssems[p][0],
                                     add=True)
                    pltpu.make_async_copy(featf_h.at[s2b], rows[p][1],
                                          gb).wait()
                    scale(rows[p][1], wb)
                    pltpu.async_copy(rows[p][1], out_sp.at[dbb], ssems[p][1],
                                     add=True)
                return carry

            lax.fori_loop(0, nch // 2, pair, 0)
            pltpu.make_async_copy(ra1, out_sp.at[dba1], ssa1).wait()
            pltpu.make_async_copy(rb1, out_sp.at[dbb1], ssb1).wait()
            pltpu.make_async_copy(featf_h.at[s2a0], ra0, sga0).wait()
            pltpu.make_async_copy(featf_h.at[s2b0], rb0, sgb0).wait()
            pltpu.make_async_copy(src_h.at[pl.ds(ebase, KCH)], srcb1,
                                  sem_is).wait()
            pltpu.make_async_copy(dst_h.at[pl.ds(ebase, KCH)], dstb1,
                                  sem_id).wait()
            pltpu.make_async_copy(wrow.at[pl.ds(ebase, KCH)], wch1,
                                  sem_w).wait()
            plsc.subcore_barrier()

            @pl.when(s < 15)
            def _():
                pltpu.sync_copy(out_sp.at[pl.ds(s * ROW_T, ROW_T)],
                                out_h.at[out_ix].at[pl.ds(s * ROW_T, ROW_T)])

            @pl.when(s == 15)
            def _():
                pltpu.sync_copy(out_sp.at[pl.ds(15 * ROW_T, ROW_LAST)],
                                out_h.at[out_ix].at[pl.ds(15 * ROW_T,
                                                          ROW_LAST)])

    return k


def _gat_layer_fused(src, dst, hp, W, al, ar, H_in, H_out, act):
    featT, el, er = _feat_el_er(hp, W, al[:, :, None], ar[:, :, None],
                                H_in, H_out)
    elT = jnp.pad(jnp.transpose(el), ((0, 0), (0, NP - N)))
    erT = jnp.pad(jnp.transpose(er), ((0, 0), (0, NP - N)))
    featf = featT.reshape(H_out * N, D)
    w_l, den = _sc_w(H_out)(src, dst, elT, erT)
    out_acc = _sc_edge2(H_out)(src, dst, w_l, featf)
    denT = jnp.transpose(den[:, :N])
    if act:
        return _norm_act(out_acc, denT, H_out)
    return out_acc, denT


def kernel(g, h, W0, al0, ar0, W1, al1, ar1, W2, al2, ar2, Wc, bc):
    src, dst = g[0], g[1]
    h0 = h.reshape(1, N, D)
    h1 = _gat_layer_fused(src, dst, h0, W0, al0, ar0, 1, 8, True)
    h2 = _gat_layer_fused(src, dst, h1, W1, al1, ar1, 8, 8, True)
    out2, den2T = _gat_layer_fused(src, dst, h2, W2, al2, ar2, 8, 1, False)
    logits, h3 = _final(out2, den2T, Wc, bc.reshape(1, -1))
    return (logits, h3)


# speedup vs baseline: 1.0012x; 1.0012x over previous
import functools

import jax
import jax.numpy as jnp
from jax import lax
from jax.experimental import pallas as pl
from jax.experimental.pallas import tpu as pltpu
from jax.experimental.pallas import tpu_sc as plsc

N = 10000
NP = 10112
D = 128
E = 320000
NEG = 0.2
BN = 400
NB = N // BN
KCH = 128
HC = 64
NTILES = 16
ROW_T = 640
ROW_LAST = N - 15 * ROW_T
EPT8 = 156 * KCH
EPT1 = 78 * KCH
EPT_W8 = 624 * KCH


def _feat_el_er(hp, W, al3, ar3, H_in, H_out):
    def body(hp_ref, w_ref, al_ref, ar_ref, feat_ref, el_ref, er_ref):
        el_cols, er_cols = [], []
        for ho in range(H_out):
            f_h = hp_ref[0] @ w_ref[0:128, ho * 128:(ho + 1) * 128]
            for hi in range(1, H_in):
                f_h = f_h + hp_ref[hi] @ w_ref[hi * 128:(hi + 1) * 128,
                                               ho * 128:(ho + 1) * 128]
            feat_ref[ho] = f_h
            el_cols.append(f_h @ al_ref[ho])
            er_cols.append(f_h @ ar_ref[ho])
        el_ref[...] = (jnp.concatenate(el_cols, axis=1)
                       if H_out > 1 else el_cols[0])
        er_ref[...] = (jnp.concatenate(er_cols, axis=1)
                       if H_out > 1 else er_cols[0])

    return pl.pallas_call(
        body,
        grid=(NB,),
        in_specs=[
            pl.BlockSpec((H_in, BN, D), lambda i: (0, i, 0)),
            pl.BlockSpec((H_in * D, H_out * D), lambda i: (0, 0)),
            pl.BlockSpec((H_out, D, 1), lambda i: (0, 0, 0)),
            pl.BlockSpec((H_out, D, 1), lambda i: (0, 0, 0)),
        ],
        out_specs=[
            pl.BlockSpec((H_out, BN, D), lambda i: (0, i, 0)),
            pl.BlockSpec((BN, H_out), lambda i: (i, 0)),
            pl.BlockSpec((BN, H_out), lambda i: (i, 0)),
        ],
        out_shape=[
            jax.ShapeDtypeStruct((H_out, N, D), jnp.float32),
            jax.ShapeDtypeStruct((N, H_out), jnp.float32),
            jax.ShapeDtypeStruct((N, H_out), jnp.float32),
        ],
    )(hp, W, al3, ar3)


def _norm_act(out_acc, denT, H):
    def body(o_ref, d_ref, y_ref):
        for h in range(H):
            dn = d_ref[:, h:h + 1]
            safe = jnp.where(dn == 0.0, 1.0, dn)
            x = o_ref[h] / safe
            y_ref[h] = jnp.where(x > 0.0, x, jnp.exp(x) - 1.0)

    return pl.pallas_call(
        body,
        grid=(NB,),
        in_specs=[
            pl.BlockSpec((H, BN, D), lambda i: (0, i, 0)),
            pl.BlockSpec((BN, H), lambda i: (i, 0)),
        ],
        out_specs=pl.BlockSpec((H, BN, D), lambda i: (0, i, 0)),
        out_shape=jax.ShapeDtypeStruct((H, N, D), jnp.float32),
    )(out_acc, denT)


def _final(out2, den2T, Wc, bc2):
    NC = Wc.shape[1]

    def body(o_ref, d_ref, wc_ref, bc_ref, log_ref, h3_ref):
        s = o_ref[0] + o_ref[1]
        dn = d_ref[:, 0:1] + d_ref[:, 1:2]
        safe = jnp.where(dn == 0.0, 1.0, dn)
        h3 = s / safe
        h3_ref[...] = h3
        log_ref[...] = h3 @ wc_ref[...] + bc_ref[...]

    return pl.pallas_call(
        body,
        grid=(NB,),
        in_specs=[
            pl.BlockSpec((2, BN, D), lambda i: (0, i, 0)),
            pl.BlockSpec((BN, 2), lambda i: (i, 0)),
            pl.BlockSpec((D, NC), lambda i: (0, 0)),
            pl.BlockSpec((1, NC), lambda i: (0, 0)),
        ],
        out_specs=[
            pl.BlockSpec((BN, NC), lambda i: (i, 0)),
            pl.BlockSpec((BN, D), lambda i: (i, 0)),
        ],
        out_shape=[
            jax.ShapeDtypeStruct((N, NC), jnp.float32),
            jax.ShapeDtypeStruct((N, D), jnp.float32),
        ],
    )(out2, den2T, Wc, bc2)


def _sc_w(Htot):
    if Htot > 1:
        heads_per_sc = Htot // 2
        n_out = Htot
    else:
        heads_per_sc = 1
        n_out = 2
    mesh = plsc.VectorSubcoreMesh(core_axis_name="c", subcore_axis_name="s")

    @functools.partial(
        pl.kernel,
        out_type=(
            jax.ShapeDtypeStruct((Htot, E), jnp.float32),
            jax.ShapeDtypeStruct((n_out, NP), jnp.float32),
        ),
        mesh=mesh,
        compiler_params=pltpu.CompilerParams(needs_layout_passes=False),
        scratch_types=[
            pltpu.VMEM((NP,), jnp.float32),
            pltpu.VMEM((NP,), jnp.float32),
            pltpu.VMEM((KCH,), jnp.int32),
            pltpu.VMEM((KCH,), jnp.int32),
            pltpu.VMEM((KCH,), jnp.int32),
            pltpu.VMEM((KCH,), jnp.int32),
            pltpu.VMEM((KCH,), jnp.float32),
            pltpu.VMEM((KCH,), jnp.float32),
            pltpu.VMEM((KCH,), jnp.int32),
            pltpu.VMEM((KCH,), jnp.int32),
            pltpu.VMEM((128,), jnp.float32),
            pltpu.VMEM_SHARED((heads_per_sc * NP,), jnp.float32),
            pltpu.SemaphoreType.DMA,
            pltpu.SemaphoreType.DMA,
            pltpu.SemaphoreType.DMA,
            pltpu.SemaphoreType.DMA,
            pltpu.SemaphoreType.DMA,
        ],
    )
    def k(src_h, dst_h, elT_h, erT_h, w_hbm, den_h,
          el_v, er_v, srcb0, srcb1, dstb0, dstb1, wb0, wb1, dn0, dn1, zd,
          den_sp, sem_is, sem_id, sem_w, sem_dn0, sem_dn1):
        c = lax.axis_index("c")
        s = lax.axis_index("s")
        wid = c * NTILES + s
        zero16 = jnp.zeros((16,), jnp.float32)
        ibufs = ((srcb0, dstb0, wb0, dn0), (srcb1, dstb1, wb1, dn1))
        if Htot > 1:
            lh = s // 4
            h_ix = c * heads_per_sc + lh
            ebase = (s % 4) * EPT_W8
            nch = jnp.where(s % 4 == 3, 628, 624)
            nzc, nzc_last = 20, 16
        else:
            lh = 0
            h_ix = 0
            ebase = wid * EPT1
            nch = jnp.where(wid == 31, 82, 78)
            nzc, nzc_last = 5, 4

        for j8 in range(8):
            zd[pl.ds(j8 * 16, 16)] = zero16

        @pl.when(s < 15)
        def _():
            for j in range(nzc):
                pltpu.sync_copy(
                    zd, den_sp.at[pl.ds((s * nzc + j) * 128, 128)])

        @pl.when(s == 15)
        def _():
            for j in range(nzc_last):
                pltpu.sync_copy(
                    zd, den_sp.at[pl.ds((15 * nzc + j) * 128, 128)])

        pltpu.sync_copy(elT_h.at[h_ix], el_v)
        pltpu.sync_copy(erT_h.at[h_ix], er_v)
        plsc.subcore_barrier()
        pltpu.sync_copy(src_h.at[pl.ds(ebase, KCH)], srcb0)
        pltpu.sync_copy(dst_h.at[pl.ds(ebase, KCH)], dstb0)
        dnoff = lh * NP

        dnsems = (sem_dn0, sem_dn1)

        def pair(ip, carry):
            for p in range(2):
                srcb, dstb, wbuf, dnb = ibufs[p]
                srcb_o, dstb_o, _, _ = ibufs[1 - p]
                sem_dn = dnsems[p]
                base = ebase + (ip * 2 + p) * KCH

                @pl.when(ip > 0)
                def _():
                    pltpu.make_async_copy(
                        wbuf, w_hbm.at[h_ix].at[pl.ds(base, KCH)],
                        sem_w).wait()
                    pltpu.make_async_copy(
                        wbuf, den_sp.at[dnb], sem_dn).wait()

                def wait_idx():
                    pltpu.make_async_copy(
                        src_h.at[pl.ds(base, KCH)], srcb, sem_is).wait()
                    pltpu.make_async_copy(
                        dst_h.at[pl.ds(base, KCH)], dstb, sem_id).wait()

                if p == 0:
                    @pl.when(ip > 0)
                    def _():
                        wait_idx()
                else:
                    wait_idx()
                nxt = jnp.minimum(base + KCH, E - KCH)
                pltpu.async_copy(src_h.at[pl.ds(nxt, KCH)], srcb_o, sem_is)
                pltpu.async_copy(dst_h.at[pl.ds(nxt, KCH)], dstb_o, sem_id)
                for j in range(KCH // 16):
                    sl = pl.ds(j * 16, 16)
                    s16 = srcb[sl]
                    d16 = dstb[sl]
                    e16 = (plsc.load_gather(el_v, [s16])
                           + plsc.load_gather(er_v, [d16]))
                    e16 = jnp.where(e16 >= 0.0, e16, e16 * NEG)
                    wbuf[sl] = jnp.exp(e16)
                    dnb[sl] = d16 + dnoff
                pltpu.async_copy(wbuf, den_sp.at[dnb], sem_dn, add=True)
                pltpu.async_copy(wbuf, w_hbm.at[h_ix].at[pl.ds(base, KCH)],
                                 sem_w)
            return carry

        lax.fori_loop(0, nch // 2, pair, 0)
        pltpu.make_async_copy(wb0, den_sp.at[dn0], sem_dn0).wait()
        pltpu.make_async_copy(wb1, den_sp.at[dn1], sem_dn1).wait()
        pltpu.make_async_copy(wb0, w_hbm.at[h_ix].at[pl.ds(ebase, KCH)],
                              sem_w).wait()
        pltpu.make_async_copy(wb1, w_hbm.at[h_ix].at[pl.ds(ebase, KCH)],
                              sem_w).wait()
        pltpu.make_async_copy(src_h.at[pl.ds(ebase, KCH)], srcb0,
                              sem_is).wait()
        pltpu.make_async_copy(dst_h.at[pl.ds(ebase, KCH)], dstb0,
                              sem_id).wait()
        plsc.subcore_barrier()

        @pl.when(s < heads_per_sc)
        def _():
            pltpu.sync_copy(den_sp.at[pl.ds(s * NP, NP)],
                            den_h.at[c * heads_per_sc + s])

    return k


def _sc_edge2(Htot):
    if Htot > 1:
        heads_per_sc = Htot // 2
        n_out = Htot
    else:
        heads_per_sc = 1
        n_out = 2
    mesh = plsc.VectorSubcoreMesh(core_axis_name="c", subcore_axis_name="s")

    @functools.partial(
        pl.kernel,
        out_type=jax.ShapeDtypeStruct((n_out, N, D), jnp.float32),
        mesh=mesh,
        compiler_params=pltpu.CompilerParams(needs_layout_passes=False),
        scratch_types=[
            pltpu.VMEM((KCH,), jnp.int32),
            pltpu.VMEM((KCH,), jnp.int32),
            pltpu.VMEM((KCH,), jnp.int32),
            pltpu.VMEM((KCH,), jnp.int32),
            pltpu.VMEM((KCH,), jnp.float32),
            pltpu.VMEM((KCH,), jnp.float32),
            pltpu.VMEM((HC,), jnp.int32),
            pltpu.VMEM((HC,), jnp.int32),
            pltpu.VMEM((HC,), jnp.int32),
            pltpu.VMEM((HC,), jnp.int32),
            pltpu.VMEM((HC,), jnp.float32),
            pltpu.VMEM((HC,), jnp.float32),
            pltpu.VMEM((HC,), jnp.int32),
            pltpu.VMEM((HC,), jnp.int32),
            pltpu.VMEM((HC,), jnp.int32),
            pltpu.VMEM((HC,), jnp.int32),
            pltpu.VMEM((HC,), jnp.float32),
            pltpu.VMEM((HC,), jnp.float32),
            pltpu.VMEM((HC, D), jnp.float32),
            pltpu.VMEM((HC, D), jnp.float32),
            pltpu.VMEM((HC, D), jnp.float32),
            pltpu.VMEM((HC, D), jnp.float32),
            pltpu.VMEM_SHARED((N, D), jnp.float32),
            pltpu.SemaphoreType.DMA,
            pltpu.SemaphoreType.DMA,
            pltpu.SemaphoreType.DMA,
            pltpu.SemaphoreType.DMA,
            pltpu.SemaphoreType.DMA,
            pltpu.SemaphoreType.DMA,
            pltpu.SemaphoreType.DMA,
            pltpu.SemaphoreType.DMA,
            pltpu.SemaphoreType.DMA,
            pltpu.SemaphoreType.DMA,
            pltpu.SemaphoreType.DMA,
        ],
    )
    def k(src_h, dst_h, w_hbm, featf_h, out_h,
          srcb0, srcb1, dstb0, dstb1, wch0, wch1,
          dba0, dbb0, s2a0, s2b0, wa0, wb0,
          dba1, dbb1, s2a1, s2b1, wa1, wb1,
          ra0, rb0, ra1, rb1, out_sp,
          sga0, sgb0, sga1, sgb1, ssa0, ssb0, ssa1, ssb1,
          sem_is, sem_id, sem_w):
        c = lax.axis_index("c")
        s = lax.axis_index("s")
        zero16 = jnp.zeros((16,), jnp.float32)
        ibufs = ((srcb0, dstb0, wch0), (srcb1, dstb1, wch1))
        bufs = ((dba0, dbb0, s2a0, s2b0, wa0, wb0),
                (dba1, dbb1, s2a1, s2b1, wa1, wb1))
        rows = ((ra0, rb0), (ra1, rb1))
        gsems = ((sga0, sgb0), (sga1, sgb1))
        ssems = ((ssa0, ssb0), (ssa1, ssb1))

        for hh in range(heads_per_sc):
            if Htot > 1:
                h_ix = c * heads_per_sc + hh
                out_ix = h_ix
                ebase = s * EPT8
                nch = jnp.where(s == 15, 160, 156)
            else:
                h_ix = 0
                out_ix = c
                ebase = (c * NTILES + s) * EPT1
                nch = jnp.where(c * NTILES + s == 31, 82, 78)
            hoff = h_ix * N
            wrow = w_hbm.at[h_ix]

            def zb_body(i, carry):
                for k8 in range(8):
                    ra0[i, pl.ds(k8 * 16, 16)] = zero16
                return carry

            lax.fori_loop(0, HC, zb_body, 0)

            @pl.when(s < 15)
            def _():
                for j in range(10):
                    pltpu.sync_copy(
                        ra0, out_sp.at[pl.ds(s * ROW_T + j * HC, HC)])

            @pl.when(s == 15)
            def _():
                for j in range(6):
                    pltpu.sync_copy(
                        ra0, out_sp.at[pl.ds(15 * ROW_T + j * HC, HC)])
                pltpu.sync_copy(ra0.at[pl.ds(0, 16)],
                                out_sp.at[pl.ds(15 * ROW_T + 384, 16)])

            def build(p):
                srcb, dstb, wch = ibufs[p]
                dba, dbb, s2a, s2b, wa, wb = bufs[p]
                for j in range(KCH // 16):
                    sl = pl.ds(j * 16, 16)
                    sl4 = pl.ds((j % 4) * 16, 16)
                    s16 = srcb[sl]
                    d16 = dstb[sl]
                    w16 = wch[sl]
                    if j < 4:
                        s2a[sl4] = s16 + hoff
                        dba[sl4] = d16
                        wa[sl4] = w16
                    else:
                        s2b[sl4] = s16 + hoff
                        dbb[sl4] = d16
                        wb[sl4] = w16

            def issue_gathers(p):
                dba, dbb, s2a, s2b, wa, wb = bufs[p]
                pltpu.async_copy(featf_h.at[s2a], rows[p][0], gsems[p][0])
                pltpu.async_copy(featf_h.at[s2b], rows[p][1], gsems[p][1])

            def prefetch(base_nxt, p):
                srcb, dstb, wch = ibufs[p]
                nxt = jnp.minimum(base_nxt, E - KCH)
                pltpu.async_copy(src_h.at[pl.ds(nxt, KCH)], srcb, sem_is)
                pltpu.async_copy(dst_h.at[pl.ds(nxt, KCH)], dstb, sem_id)
                pltpu.async_copy(wrow.at[pl.ds(nxt, KCH)], wch, sem_w)

            def scale(rbuf, wbuf):
                def sbody(g2, carry2):
                    w16 = wbuf[pl.ds(g2 * 16, 16)]
                    for e in range(16):
                        r = g2 * 16 + e
                        ws = w16[e]
                        for k8 in range(8):
                            csl = pl.ds(k8 * 16, 16)
                            rbuf[r, csl] = rbuf[r, csl] * ws
                    return carry2

                lax.fori_loop(0, HC // 16, sbody, 0)

            pltpu.sync_copy(src_h.at[pl.ds(ebase, KCH)], srcb0)
            pltpu.sync_copy(dst_h.at[pl.ds(ebase, KCH)], dstb0)
            pltpu.sync_copy(wrow.at[pl.ds(ebase, KCH)], wch0)
            build(0)
            issue_gathers(0)
            prefetch(ebase + KCH, 1)
            plsc.subcore_barrier()

            def pair(ip, carry):
                for p in range(2):
                    q = 1 - p
                    base = ebase + (ip * 2 + p) * KCH
                    dba, dbb, s2a, s2b, wa, wb = bufs[p]
                    dbao, dbbo, _, _, _, _ = bufs[q]
                    srcb_q, dstb_q, wch_q = ibufs[q]

                    nb = base + KCH
                    nbc = jnp.minimum(nb, E - KCH)
                    pltpu.make_async_copy(
                        src_h.at[pl.ds(nbc, KCH)], srcb_q, sem_is).wait()
                    pltpu.make_async_copy(
                        dst_h.at[pl.ds(nbc, KCH)], dstb_q, sem_id).wait()
                    pltpu.make_async_copy(
                        wrow.at[pl.ds(nbc, KCH)], wch_q, sem_w).wait()

                    def wait_sc_prev():
                        pltpu.make_async_copy(
                            rows[q][0], out_sp.at[dbao], ssems[q][0]).wait()
                        pltpu.make_async_copy(
                            rows[q][1], out_sp.at[dbbo], ssems[q][1]).wait()

                    if p == 0:
                        @pl.when(ip > 0)
                        def _():
                            wait_sc_prev()
                    else:
                        wait_sc_prev()
                    build(q)
                    issue_gathers(q)
                    prefetch(nb + KCH, p)
                    ga, gb = gsems[p]
                    pltpu.make_async_copy(featf_h.at[s2a], rows[p][0],
                                          ga).wait()
                    scale(rows[p][0], wa)
                    pltpu.async_copy(rows[p][0], out_sp.at[dba], ssems[p][0],
                                     add=True)
                    pltpu.make_async_copy(featf_h.at[s2b], rows[p][1],
                                          gb).wait()
                    scale(rows[p][1], wb)
                    pltpu.async_copy(rows[p][1], out_sp.at[dbb], ssems[p][1],
                                     add=True)
                return carry

            lax.fori_loop(0, nch // 2, pair, 0)
            pltpu.make_async_copy(ra1, out_sp.at[dba1], ssa1).wait()
            pltpu.make_async_copy(rb1, out_sp.at[dbb1], ssb1).wait()
            pltpu.make_async_copy(featf_h.at[s2a0], ra0, sga0).wait()
            pltpu.make_async_copy(featf_h.at[s2b0], rb0, sgb0).wait()
            pltpu.make_async_copy(src_h.at[pl.ds(ebase, KCH)], srcb1,
                                  sem_is).wait()
            pltpu.make_async_copy(dst_h.at[pl.ds(ebase, KCH)], dstb1,
                                  sem_id).wait()
            pltpu.make_async_copy(wrow.at[pl.ds(ebase, KCH)], wch1,
                                  sem_w).wait()
            plsc.subcore_barrier()

            @pl.when(s < 15)
            def _():
                pltpu.sync_copy(out_sp.at[pl.ds(s * ROW_T, ROW_T)],
                                out_h.at[out_ix].at[pl.ds(s * ROW_T, ROW_T)])

            @pl.when(s == 15)
            def _():
                pltpu.sync_copy(out_sp.at[pl.ds(15 * ROW_T, ROW_LAST)],
                                out_h.at[out_ix].at[pl.ds(15 * ROW_T,
                                                          ROW_LAST)])

    return k


def _gat_layer_fused(src, dst, hp, W, al, ar, H_in, H_out, act):
    featT, el, er = _feat_el_er(hp, W, al[:, :, None], ar[:, :, None],
                                H_in, H_out)
    elT = jnp.pad(jnp.transpose(el), ((0, 0), (0, NP - N)))
    erT = jnp.pad(jnp.transpose(er), ((0, 0), (0, NP - N)))
    featf = featT.reshape(H_out * N, D)
    w_l, den = _sc_w(H_out)(src, dst, elT, erT)
    out_acc = _sc_edge2(H_out)(src, dst, w_l, featf)
    denT = jnp.transpose(den[:, :N])
    if act:
        return _norm_act(out_acc, denT, H_out)
    return out_acc, denT


def kernel(g, h, W0, al0, ar0, W1, al1, ar1, W2, al2, ar2, Wc, bc):
    src, dst = g[0], g[1]
    h0 = h.reshape(1, N, D)
    h1 = _gat_layer_fused(src, dst, h0, W0, al0, ar0, 1, 8, True)
    h2 = _gat_layer_fused(src, dst, h1, W1, al1, ar1, 8, 8, True)
    out2, den2T = _gat_layer_fused(src, dst, h2, W2, al2, ar2, 8, 1, False)
    logits, h3 = _final(out2, den2T, Wc, bc.reshape(1, -1))
    return (logits, h3)
